# Initial kernel scaffold; baseline (speedup 1.0000x reference)
import functools

import jax
import jax.numpy as jnp
from jax import lax
from jax.experimental import pallas as pl
from jax.experimental.pallas import tpu as pltpu
from jax.experimental.pallas import tpu_sc as plsc

N = 50000
D = 32
E = 1600000

TILES = 16
NPAD = 50048
RSLICE = NPAD // TILES
QCH = RSLICE // 8

RPT = 784
ROWS = TILES * RPT
EPAD = ROWS * 128
CH = 8
NBLK = RPT // CH

_MESH = plsc.VectorSubcoreMesh(
    core_axis_name="c", subcore_axis_name="s", num_cores=2, num_subcores=TILES
)


@functools.partial(
    pl.kernel,
    out_type=(
        jax.ShapeDtypeStruct((NPAD,), jnp.float32),
        jax.ShapeDtypeStruct((NPAD,), jnp.float32),
    ),
    mesh=_MESH,
    scratch_types=[
        pltpu.VMEM((CH, 128), jnp.int32),
        pltpu.VMEM((128,), jnp.float32),
        pltpu.VMEM((RSLICE,), jnp.float32),
        pltpu.VMEM_SHARED((NPAD,), jnp.float32),
    ],
)
def _deg_kernel(src_hbm, dst_hbm, zeros1_hbm, ones_hbm,
                degu_hbm, degi_hbm, idx_v, ones_v, stage_v, acc):
    cid = lax.axis_index("c")
    sid = lax.axis_index("s")
    o0 = sid * RSLICE
    pltpu.sync_copy(ones_hbm, ones_v)
    pltpu.sync_copy(zeros1_hbm.at[pl.ds(o0, RSLICE)], acc.at[pl.ds(o0, RSLICE)])
    plsc.subcore_barrier()

    def run(idx_hbm):
        def blk(b, carry):
            r0 = sid * RPT + b * CH
            pltpu.sync_copy(idx_hbm.at[pl.ds(r0, CH)], idx_v)
            for j in range(CH):
                pltpu.sync_copy(ones_v, acc.at[idx_v.at[j]], add=True)
            return carry
        lax.fori_loop(0, NBLK, blk, 0)

    @pl.when(cid == 0)
    def _():
        run(src_hbm)

    @pl.when(cid == 1)
    def _():
        run(dst_hbm)

    plsc.subcore_barrier()

    def wout(out_hbm):
        pltpu.sync_copy(acc.at[pl.ds(o0, RSLICE)], stage_v)
        pltpu.sync_copy(stage_v, out_hbm.at[pl.ds(o0, RSLICE)])

    @pl.when(cid == 0)
    def _():
        wout(degu_hbm)

    @pl.when(cid == 1)
    def _():
        wout(degi_hbm)


@functools.partial(
    pl.kernel,
    out_type=(
        jax.ShapeDtypeStruct((NPAD, D), jnp.float32),
        jax.ShapeDtypeStruct((NPAD, D), jnp.float32),
    ),
    mesh=_MESH,
    scratch_types=[
        pltpu.VMEM((CH, 128), jnp.int32),
        pltpu.VMEM((CH, 128), jnp.int32),
        pltpu.VMEM((CH * 128, D), jnp.float32),
        pltpu.VMEM((QCH, D), jnp.float32),
        pltpu.VMEM_SHARED((NPAD, D), jnp.float32),
        pltpu.SemaphoreType.DMA,
    ],
)
def _layer_kernel(yu_hbm, yi_hbm, src_hbm, dst_hbm, zeros2_hbm,
                  si_hbm, su_hbm, gidx_v, sidx_v, rows_v, stage_v, acc, sem):
    cid = lax.axis_index("c")
    sid = lax.axis_index("s")
    o0 = sid * RSLICE
    pltpu.sync_copy(zeros2_hbm.at[pl.ds(o0, RSLICE)], acc.at[pl.ds(o0, RSLICE)])
    plsc.subcore_barrier()

    def run(tab_hbm, g_hbm, s_hbm):
        def blk(b, carry):
            r0 = sid * RPT + b * CH
            pltpu.sync_copy(g_hbm.at[pl.ds(r0, CH)], gidx_v)
            pltpu.sync_copy(s_hbm.at[pl.ds(r0, CH)], sidx_v)
            descs = [
                pltpu.async_copy(
                    tab_hbm.at[gidx_v.at[j]],
                    rows_v.at[pl.ds(j * 128, 128)],
                    sem,
                )
                for j in range(CH)
            ]
            for d in descs:
                d.wait()
            for j in range(CH):
                pltpu.sync_copy(
                    rows_v.at[pl.ds(j * 128, 128)],
                    acc.at[sidx_v.at[j]],
                    add=True,
                )
            return carry
        lax.fori_loop(0, NBLK, blk, 0)

    @pl.when(cid == 0)
    def _():
        run(yu_hbm, src_hbm, dst_hbm)

    @pl.when(cid == 1)
    def _():
        run(yi_hbm, dst_hbm, src_hbm)

    plsc.subcore_barrier()

    def wout(out_hbm):
        for q in range(8):
            q0 = o0 + q * QCH
            pltpu.sync_copy(acc.at[pl.ds(q0, QCH)], stage_v)
            pltpu.sync_copy(stage_v, out_hbm.at[pl.ds(q0, QCH)])

    @pl.when(cid == 0)
    def _():
        wout(si_hbm)

    @pl.when(cid == 1)
    def _():
        wout(su_hbm)


_GRID = 16
_BR = NPAD // _GRID


def _node_spec(width):
    return pl.BlockSpec((_BR, width), lambda i: (i, 0))


def _prep_body(du, di, ut, it, au, ai, yu, yi):
    a_u = lax.rsqrt(jnp.maximum(du[...], 1.0))
    a_i = lax.rsqrt(jnp.maximum(di[...], 1.0))
    au[...] = a_u
    ai[...] = a_i
    yu[...] = ut[...] * a_u
    yi[...] = it[...] * a_i


_prep_call = pl.pallas_call(
    _prep_body,
    grid=(_GRID,),
    in_specs=[_node_spec(1), _node_spec(1), _node_spec(D), _node_spec(D)],
    out_specs=[_node_spec(1), _node_spec(1), _node_spec(D), _node_spec(D)],
    out_shape=[
        jax.ShapeDtypeStruct((NPAD, 1), jnp.float32),
        jax.ShapeDtypeStruct((NPAD, 1), jnp.float32),
        jax.ShapeDtypeStruct((NPAD, D), jnp.float32),
        jax.ShapeDtypeStruct((NPAD, D), jnp.float32),
    ],
)


def _mid_body(au, ai, su, si, yu, yi):
    yu[...] = au[...] * au[...] * su[...]
    yi[...] = ai[...] * ai[...] * si[...]


_mid_call = pl.pallas_call(
    _mid_body,
    grid=(_GRID,),
    in_specs=[_node_spec(1), _node_spec(1), _node_spec(D), _node_spec(D)],
    out_specs=[_node_spec(D), _node_spec(D)],
    out_shape=[
        jax.ShapeDtypeStruct((NPAD, D), jnp.float32),
        jax.ShapeDtypeStruct((NPAD, D), jnp.float32),
    ],
)


def _fin_body(ut, au, su1, su2, it, ai, si1, si2, eu, ei):
    third = jnp.float32(1.0 / 3.0)
    eu[...] = (ut[...] + au[...] * (su1[...] + su2[...])) * third
    ei[...] = (it[...] + ai[...] * (si1[...] + si2[...])) * third


_fin_call = pl.pallas_call(
    _fin_body,
    grid=(_GRID,),
    in_specs=[
        _node_spec(D), _node_spec(1), _node_spec(D), _node_spec(D),
        _node_spec(D), _node_spec(1), _node_spec(D), _node_spec(D),
    ],
    out_specs=[_node_spec(D), _node_spec(D)],
    out_shape=[
        jax.ShapeDtypeStruct((NPAD, D), jnp.float32),
        jax.ShapeDtypeStruct((NPAD, D), jnp.float32),
    ],
)


def kernel(user_table, item_table, user_ids, item_ids, edge_index):
    f32 = jnp.float32
    src = edge_index[0]
    dst = edge_index[1]
    pad_idx = jnp.full((EPAD - E,), N, dtype=jnp.int32)
    src2 = jnp.concatenate([src, pad_idx]).reshape(ROWS, 128)
    dst2 = jnp.concatenate([dst, pad_idx]).reshape(ROWS, 128)

    zpad = jnp.zeros((NPAD - N, D), dtype=f32)
    utab = jnp.concatenate([user_table, zpad], axis=0)
    itab = jnp.concatenate([item_table, zpad], axis=0)

    z1 = jnp.zeros((NPAD,), dtype=f32)
    z2 = jnp.zeros((NPAD, D), dtype=f32)
    ones = jnp.ones((128,), dtype=f32)

    deg_u, deg_i = _deg_kernel(src2, dst2, z1, ones)
    a_u, a_i, yu0, yi0 = _prep_call(
        deg_u.reshape(NPAD, 1), deg_i.reshape(NPAD, 1), utab, itab
    )
    s_i1, s_u1 = _layer_kernel(yu0, yi0, src2, dst2, z2)
    yu1, yi1 = _mid_call(a_u, a_i, s_u1, s_i1)
    s_i2, s_u2 = _layer_kernel(yu1, yi1, src2, dst2, z2)
    emb_u, emb_i = _fin_call(utab, a_u, s_u1, s_u2, itab, a_i, s_i1, s_i2)
    return jnp.concatenate([emb_u[:N], emb_i[:N]], axis=0)


# speedup vs baseline: 29.1928x; 29.1928x over previous
import functools

import jax
import jax.numpy as jnp
from jax import lax
from jax.experimental import pallas as pl
from jax.experimental.pallas import tpu as pltpu
from jax.experimental.pallas import tpu_sc as plsc

N = 50000
D = 32
E = 1600000

TILES = 16
NPAD = 50048
RSLICE = NPAD // TILES
QCH = RSLICE // 8

RPT = 784
ROWS = TILES * RPT
EPAD = ROWS * 128
CH = 7
NBLK = RPT // CH

_MESH = plsc.VectorSubcoreMesh(
    core_axis_name="c", subcore_axis_name="s", num_cores=2, num_subcores=TILES
)


@functools.partial(
    pl.kernel,
    out_type=(
        jax.ShapeDtypeStruct((NPAD,), jnp.float32),
        jax.ShapeDtypeStruct((NPAD,), jnp.float32),
    ),
    mesh=_MESH,
    scratch_types=[
        pltpu.VMEM((CH, 128), jnp.int32),
        pltpu.VMEM((128,), jnp.float32),
        pltpu.VMEM((RSLICE,), jnp.float32),
        pltpu.VMEM_SHARED((NPAD,), jnp.float32),
    ],
    compiler_params=pltpu.CompilerParams(use_tc_tiling_on_sc=False),
)
def _deg_kernel(src_hbm, dst_hbm, zeros1_hbm, ones_hbm,
                degu_hbm, degi_hbm, idx_v, ones_v, stage_v, acc):
    cid = lax.axis_index("c")
    sid = lax.axis_index("s")
    o0 = sid * RSLICE
    pltpu.sync_copy(ones_hbm, ones_v)
    pltpu.sync_copy(zeros1_hbm.at[pl.ds(o0, RSLICE)], stage_v)
    pltpu.sync_copy(stage_v, acc.at[pl.ds(o0, RSLICE)])
    plsc.subcore_barrier()

    def run(idx_hbm):
        def blk(b, carry):
            r0 = sid * RPT + b * CH
            pltpu.sync_copy(idx_hbm.at[pl.ds(r0, CH)], idx_v)
            for j in range(CH):
                pltpu.sync_copy(ones_v, acc.at[idx_v.at[j]], add=True)
            return carry
        lax.fori_loop(0, NBLK, blk, 0)

    @pl.when(cid == 0)
    def _():
        run(src_hbm)

    @pl.when(cid == 1)
    def _():
        run(dst_hbm)

    plsc.subcore_barrier()

    def wout(out_hbm):
        pltpu.sync_copy(acc.at[pl.ds(o0, RSLICE)], stage_v)
        pltpu.sync_copy(stage_v, out_hbm.at[pl.ds(o0, RSLICE)])

    @pl.when(cid == 0)
    def _():
        wout(degu_hbm)

    @pl.when(cid == 1)
    def _():
        wout(degi_hbm)


_WCH = (896, 896, 896, 440)


@functools.partial(
    pl.kernel,
    out_type=(
        jax.ShapeDtypeStruct((NPAD, D), jnp.float32),
        jax.ShapeDtypeStruct((NPAD, D), jnp.float32),
    ),
    mesh=_MESH,
    scratch_types=[
        pltpu.VMEM((CH, 128), jnp.int32),
        pltpu.VMEM((CH, 128), jnp.int32),
        pltpu.VMEM((CH * 128, D), jnp.float32),
        pltpu.VMEM_SHARED((NPAD, D), jnp.float32),
        pltpu.SemaphoreType.DMA,
    ],
    compiler_params=pltpu.CompilerParams(use_tc_tiling_on_sc=False),
)
def _layer_kernel(yu_hbm, yi_hbm, src_hbm, dst_hbm, zeros2_hbm,
                  si_hbm, su_hbm, gidx_v, sidx_v, rows_v, acc, sem):
    cid = lax.axis_index("c")
    sid = lax.axis_index("s")
    o0 = sid * RSLICE
    q0 = o0
    for w in _WCH:
        pltpu.sync_copy(zeros2_hbm.at[pl.ds(q0, w)], rows_v.at[pl.ds(0, w)])
        pltpu.sync_copy(rows_v.at[pl.ds(0, w)], acc.at[pl.ds(q0, w)])
        q0 += w
    plsc.subcore_barrier()

    def run(tab_hbm, g_hbm, s_hbm):
        def blk(b, carry):
            r0 = sid * RPT + b * CH
            pltpu.sync_copy(g_hbm.at[pl.ds(r0, CH)], gidx_v)
            pltpu.sync_copy(s_hbm.at[pl.ds(r0, CH)], sidx_v)
            descs = [
                pltpu.async_copy(
                    tab_hbm.at[gidx_v.at[j]],
                    rows_v.at[pl.ds(j * 128, 128)],
                    sem,
                )
                for j in range(CH)
            ]
            for d in descs:
                d.wait()
            for j in range(CH):
                pltpu.sync_copy(
                    rows_v.at[pl.ds(j * 128, 128)],
                    acc.at[sidx_v.at[j]],
                    add=True,
                )
            return carry
        lax.fori_loop(0, NBLK, blk, 0)

    @pl.when(cid == 0)
    def _():
        run(yu_hbm, src_hbm, dst_hbm)

    @pl.when(cid == 1)
    def _():
        run(yi_hbm, dst_hbm, src_hbm)

    plsc.subcore_barrier()

    def wout(out_hbm):
        p0 = o0
        for w in _WCH:
            pltpu.sync_copy(acc.at[pl.ds(p0, w)], rows_v.at[pl.ds(0, w)])
            pltpu.sync_copy(rows_v.at[pl.ds(0, w)], out_hbm.at[pl.ds(p0, w)])
            p0 += w

    @pl.when(cid == 0)
    def _():
        wout(si_hbm)

    @pl.when(cid == 1)
    def _():
        wout(su_hbm)


_GRID = 16
_BR = NPAD // _GRID


def _node_spec(width):
    return pl.BlockSpec((_BR, width), lambda i: (i, 0))


def _prep_body(du, di, ut, it, au, ai, yu, yi):
    a_u = lax.rsqrt(jnp.maximum(du[...], 1.0))
    a_i = lax.rsqrt(jnp.maximum(di[...], 1.0))
    au[...] = a_u
    ai[...] = a_i
    yu[...] = ut[...] * a_u
    yi[...] = it[...] * a_i


_prep_call = pl.pallas_call(
    _prep_body,
    grid=(_GRID,),
    in_specs=[_node_spec(1), _node_spec(1), _node_spec(D), _node_spec(D)],
    out_specs=[_node_spec(1), _node_spec(1), _node_spec(D), _node_spec(D)],
    out_shape=[
        jax.ShapeDtypeStruct((NPAD, 1), jnp.float32),
        jax.ShapeDtypeStruct((NPAD, 1), jnp.float32),
        jax.ShapeDtypeStruct((NPAD, D), jnp.float32),
        jax.ShapeDtypeStruct((NPAD, D), jnp.float32),
    ],
)


def _mid_body(au, ai, su, si, yu, yi):
    yu[...] = au[...] * au[...] * su[...]
    yi[...] = ai[...] * ai[...] * si[...]


_mid_call = pl.pallas_call(
    _mid_body,
    grid=(_GRID,),
    in_specs=[_node_spec(1), _node_spec(1), _node_spec(D), _node_spec(D)],
    out_specs=[_node_spec(D), _node_spec(D)],
    out_shape=[
        jax.ShapeDtypeStruct((NPAD, D), jnp.float32),
        jax.ShapeDtypeStruct((NPAD, D), jnp.float32),
    ],
)


def _fin_body(ut, au, su1, su2, it, ai, si1, si2, eu, ei):
    third = jnp.float32(1.0 / 3.0)
    eu[...] = (ut[...] + au[...] * (su1[...] + su2[...])) * third
    ei[...] = (it[...] + ai[...] * (si1[...] + si2[...])) * third


_fin_call = pl.pallas_call(
    _fin_body,
    grid=(_GRID,),
    in_specs=[
        _node_spec(D), _node_spec(1), _node_spec(D), _node_spec(D),
        _node_spec(D), _node_spec(1), _node_spec(D), _node_spec(D),
    ],
    out_specs=[_node_spec(D), _node_spec(D)],
    out_shape=[
        jax.ShapeDtypeStruct((NPAD, D), jnp.float32),
        jax.ShapeDtypeStruct((NPAD, D), jnp.float32),
    ],
)


def kernel(user_table, item_table, user_ids, item_ids, edge_index):
    f32 = jnp.float32
    src = edge_index[0]
    dst = edge_index[1]
    pad_idx = jnp.full((EPAD - E,), N, dtype=jnp.int32)
    src2 = jnp.concatenate([src, pad_idx]).reshape(ROWS, 128)
    dst2 = jnp.concatenate([dst, pad_idx]).reshape(ROWS, 128)

    zpad = jnp.zeros((NPAD - N, D), dtype=f32)
    utab = jnp.concatenate([user_table, zpad], axis=0)
    itab = jnp.concatenate([item_table, zpad], axis=0)

    z1 = jnp.zeros((NPAD,), dtype=f32)
    z2 = jnp.zeros((NPAD, D), dtype=f32)
    ones = jnp.ones((128,), dtype=f32)

    deg_u, deg_i = _deg_kernel(src2, dst2, z1, ones)
    a_u, a_i, yu0, yi0 = _prep_call(
        deg_u.reshape(NPAD, 1), deg_i.reshape(NPAD, 1), utab, itab
    )
    s_i1, s_u1 = _layer_kernel(yu0, yi0, src2, dst2, z2)
    yu1, yi1 = _mid_call(a_u, a_i, s_u1, s_i1)
    s_i2, s_u2 = _layer_kernel(yu1, yi1, src2, dst2, z2)
    emb_u, emb_i = _fin_call(utab, a_u, s_u1, s_u2, itab, a_i, s_i1, s_i2)
    return jnp.concatenate([emb_u[:N], emb_i[:N]], axis=0)
